# Initial kernel scaffold; baseline (speedup 1.0000x reference)
import jax
import jax.numpy as jnp
from jax.experimental import pallas as pl


def kernel(x, edge_index, edge_attr, batch, Wc1, bc1, Wc2, bc2, Wc3, bc3, Wg1, as1, ad1, bg1, Wg2, as2, ad2, bg2, Wg3, as3, ad3, bg3, Wi1a, bi1a, Wi1b, bi1b, Wi2a, bi2a, Wi2b, bi2b, Wi3a, bi3a, Wi3b, bi3b, Wf1, bf1, Wf2, bf2, Wf3, bf3):
    raise NotImplementedError("write your pallas kernel here")


# speedup vs baseline: 1.0468x; 1.0468x over previous
import jax
import jax.numpy as jnp
from jax.experimental import pallas as pl

_G = 128
_NC = 10
_EPS = 1e-5


def _row_block(rows):
    return 1000 if rows % 1000 == 0 else rows


def _mm(X, W, b=None, relu=False, exact=False):
    rows, K = X.shape
    Fo = W.shape[1]
    BR = _row_block(rows)
    if b is None:
        b = jnp.zeros((Fo,), X.dtype)
    b2 = b.reshape(1, Fo)

    def kfn(x_ref, w_ref, b_ref, o_ref):
        xb = x_ref[...]
        wb = w_ref[...]
        if exact:
            y = jnp.dot(xb, wb, preferred_element_type=jnp.float32,
                        precision=jax.lax.Precision.HIGHEST)
        else:
            y = jnp.dot(xb.astype(jnp.bfloat16), wb.astype(jnp.bfloat16),
                        preferred_element_type=jnp.float32)
        y = y + b_ref[...]
        if relu:
            y = jnp.maximum(y, 0.0)
        o_ref[...] = y

    return pl.pallas_call(
        kfn,
        grid=(rows // BR,),
        in_specs=[
            pl.BlockSpec((BR, K), lambda i: (i, 0)),
            pl.BlockSpec((K, Fo), lambda i: (0, 0)),
            pl.BlockSpec((1, Fo), lambda i: (0, 0)),
        ],
        out_specs=pl.BlockSpec((BR, Fo), lambda i: (i, 0)),
        out_shape=jax.ShapeDtypeStruct((rows, Fo), X.dtype),
    )(X, W, b2)


def _post(Y, b):
    rows, Fo = Y.shape
    BR = _row_block(rows)
    b2 = b.reshape(1, Fo)

    def kfn(y_ref, b_ref, r_ref, s_ref):
        i = pl.program_id(0)

        @pl.when(i == 0)
        def _():
            s_ref[...] = jnp.zeros_like(s_ref)

        r = jnp.maximum(y_ref[...] + b_ref[...], 0.0)
        r_ref[...] = r
        su = jnp.sum(r, axis=0, keepdims=True)
        sq = jnp.sum(r * r, axis=0, keepdims=True)
        pad = jnp.zeros((6, r.shape[1]), r.dtype)
        s_ref[...] = s_ref[...] + jnp.concatenate([su, sq, pad], axis=0)

    R, S = pl.pallas_call(
        kfn,
        grid=(rows // BR,),
        in_specs=[
            pl.BlockSpec((BR, Fo), lambda i: (i, 0)),
            pl.BlockSpec((1, Fo), lambda i: (0, 0)),
        ],
        out_specs=[
            pl.BlockSpec((BR, Fo), lambda i: (i, 0)),
            pl.BlockSpec((8, Fo), lambda i: (0, 0)),
        ],
        out_shape=[
            jax.ShapeDtypeStruct((rows, Fo), Y.dtype),
            jax.ShapeDtypeStruct((8, Fo), Y.dtype),
        ],
    )(Y, b2)
    return R, S


def _scale(Y, sc, sh):
    rows, Fo = Y.shape
    BR = _row_block(rows)

    def kfn(y_ref, a_ref, c_ref, o_ref):
        o_ref[...] = y_ref[...] * a_ref[...] + c_ref[...]

    return pl.pallas_call(
        kfn,
        grid=(rows // BR,),
        in_specs=[
            pl.BlockSpec((BR, Fo), lambda i: (i, 0)),
            pl.BlockSpec((1, Fo), lambda i: (0, 0)),
            pl.BlockSpec((1, Fo), lambda i: (0, 0)),
        ],
        out_specs=pl.BlockSpec((BR, Fo), lambda i: (i, 0)),
        out_shape=jax.ShapeDtypeStruct((rows, Fo), Y.dtype),
    )(Y, sc, sh)


def _bn_apply(R, S, rows):
    m = (S[0:1] / rows)
    var = S[1:2] / rows - m * m
    rs = 1.0 / jnp.sqrt(var + _EPS)
    return _scale(R, rs, -m * rs)


def _relu_bn(Y, b):
    R, S = _post(Y, b)
    return _bn_apply(R, S, Y.shape[0])


def _pool(h, batch):
    rows, Fo = h.shape
    BR = _row_block(rows)
    b2 = batch.reshape(rows, 1)

    def kfn(h_ref, b_ref, o_ref):
        i = pl.program_id(0)

        @pl.when(i == 0)
        def _():
            o_ref[...] = jnp.zeros_like(o_ref)

        seg = jax.lax.broadcasted_iota(jnp.int32, (BR, _G), 1)
        onehot = (b_ref[...] == seg).astype(h_ref.dtype)
        o_ref[...] = o_ref[...] + jax.lax.dot_general(
            onehot, h_ref[...], (((0,), (0,)), ((), ())),
            preferred_element_type=jnp.float32,
            precision=jax.lax.Precision.HIGHEST)

    return pl.pallas_call(
        kfn,
        grid=(rows // BR,),
        in_specs=[
            pl.BlockSpec((BR, Fo), lambda i: (i, 0)),
            pl.BlockSpec((BR, 1), lambda i: (i, 0)),
        ],
        out_specs=pl.BlockSpec((_G, Fo), lambda i: (0, 0)),
        out_shape=jax.ShapeDtypeStruct((_G, Fo), h.dtype),
    )(h, b2)


def kernel(x, edge_index, edge_attr, batch, Wc1, bc1, Wc2, bc2, Wc3, bc3, Wg1, as1, ad1, bg1, Wg2, as2, ad2, bg2, Wg3, as3, ad3, bg3, Wi1a, bi1a, Wi1b, bi1b, Wi2a, bi2a, Wi2b, bi2b, Wi3a, bi3a, Wi3b, bi3b, Wf1, bf1, Wf2, bf2, Wf3, bf3):
    n = x.shape[0]
    s = edge_index[0]
    d = edge_index[1]
    loop = jnp.arange(n, dtype=s.dtype)
    s2 = jnp.concatenate([s, loop])
    d2 = jnp.concatenate([d, loop])

    deg = jnp.zeros((n,), x.dtype).at[d2].add(1.0)
    dinv = 1.0 / jnp.sqrt(jnp.maximum(deg, 1e-12))
    norm = dinv[s2] * dinv[d2]

    def gcn(h, W, b):
        xw = _mm(h, W)
        agg = jnp.zeros_like(xw).at[d2].add(xw[s2] * norm[:, None])
        return _relu_bn(agg, b)

    def gat(h, W, att_s, att_d, bias, H, C):
        xw = _mm(h, W)
        A = jnp.zeros((H * C, 2 * H), x.dtype)
        for hh in range(H):
            A = A.at[hh * C:(hh + 1) * C, hh].set(att_s[hh])
            A = A.at[hh * C:(hh + 1) * C, H + hh].set(att_d[hh])
        aa = _mm(xw, A, exact=True)
        a_s, a_d = aa[:, :H], aa[:, H:]
        alpha = a_s[s2] + a_d[d2]
        alpha = jnp.where(alpha >= 0, alpha, 0.2 * alpha)
        ex = jnp.exp(alpha)
        den = jnp.zeros((n, H), x.dtype).at[d2].add(ex)
        coef = ex / den[d2]
        msg = xw[s2].reshape(-1, H, C) * coef[:, :, None]
        out = jnp.zeros((n, H, C), x.dtype).at[d2].add(msg).reshape(n, H * C)
        return _relu_bn(out, bias)

    def gin(h, W1, b1, W2, b2):
        agg = jnp.zeros_like(h).at[d].add(h[s])
        t = _mm(h + agg, W1, b1, relu=True)
        t = _mm(t, W2)
        return _relu_bn(t, b2)

    h = gcn(x, Wc1, bc1)
    h = gcn(h, Wc2, bc2)
    h = gcn(h, Wc3, bc3)
    hx = _pool(h, batch)

    y = gat(x, Wg1, as1, ad1, bg1, 8, 128)
    y = gat(y, Wg2, as2, ad2, bg2, 8, 64)
    y = gat(y, Wg3, as3, ad3, bg3, 8, 32)
    hy = _pool(y, batch)

    z = gin(x, Wi1a, bi1a, Wi1b, bi1b)
    z = gin(z, Wi2a, bi2a, Wi2b, bi2b)
    z = gin(z, Wi3a, bi3a, Wi3b, bi3b)
    hz = _pool(z, batch)

    cr = jnp.concatenate([hx, hy, hz], axis=1)
    cr = _mm(cr, Wf1, bf1, relu=True)
    cr = _mm(cr, Wf2, bf2, relu=True)
    cr = _mm(cr, Wf3, bf3)
    return cr.reshape(-1, _NC)
